# baseline (speedup 1.0000x reference)
import functools

import jax
import jax.numpy as jnp
from jax.experimental import pallas as pl
from jax.experimental.pallas import tpu as pltpu

B = 16384
K = 1024
D = 512
ALPHA = 1.0

BM = 1024
NB = B // BM
TM = 256
NT = BM // TM


def _loss_kernel(emb_ref, lab_ref, cent_ref, out_ref, cn_ref, acc_ref):
    i = pl.program_id(0)

    @pl.when(i == 0)
    def _init():
        c = cent_ref[...]
        cs = jnp.sum(c * c, axis=1, keepdims=True)
        inv = -2.0 / jnp.maximum(jnp.sqrt(cs), 1e-12)
        cn_ref[...] = (c * inv).T
        acc_ref[...] = jnp.zeros((3, TM), jnp.float32)

    cn = cn_ref[...]
    acc_b = acc_ref[0, :]
    acc_s = acc_ref[1, :]
    acc_n = acc_ref[2, :]
    for t in range(NT):
        e = emb_ref[t * TM:(t + 1) * TM, :]
        dot = jax.lax.dot_general(
            e, cn, (((1,), (0,)), ((), ())),
            preferred_element_type=jnp.float32,
        )
        m = jnp.min(dot, axis=1)
        es = jnp.sum(e * e, axis=1)
        inv_e = 1.0 / jnp.maximum(jnp.sqrt(es), 1e-12)
        min_d2 = jnp.maximum(2.0 + m * inv_e, 1e-12)

        lab = lab_ref[t * TM:(t + 1) * TM]
        bona = lab == 0
        acc_b = acc_b + jnp.where(bona, min_d2, 0.0)
        acc_s = acc_s + jnp.where(bona, 0.0, min_d2)
        acc_n = acc_n + lab.astype(jnp.float32)
    acc_ref[0, :] = acc_b
    acc_ref[1, :] = acc_s
    acc_ref[2, :] = acc_n

    @pl.when(i == NB - 1)
    def _finalize():
        n_spoof = jnp.sum(acc_ref[2, :])
        n_bona = float(B) - n_spoof
        bona_loss = jnp.sum(acc_ref[0, :]) / jnp.maximum(n_bona, 1.0)
        spoof_loss = -ALPHA * (jnp.sum(acc_ref[1, :]) / jnp.maximum(n_spoof, 1.0))
        total = (jnp.where(n_bona > 0.0, bona_loss, 0.0)
                 + jnp.where(n_spoof > 0.0, spoof_loss, 0.0))
        out_ref[0, 0] = total


@functools.partial(jax.jit, static_argnames=("interpret",))
def kernel(embeddings, labels, bonafide_centers, interpret=False):
    out = pl.pallas_call(
        _loss_kernel,
        grid=(NB,),
        in_specs=[
            pl.BlockSpec((BM, D), lambda i: (i, 0)),
            pl.BlockSpec((BM,), lambda i: (i,)),
            pl.BlockSpec((K, D), lambda i: (0, 0)),
        ],
        out_specs=pl.BlockSpec(memory_space=pltpu.SMEM),
        out_shape=jax.ShapeDtypeStruct((1, 1), jnp.float32),
        scratch_shapes=[pltpu.VMEM((D, K), jnp.float32),
                        pltpu.VMEM((3, TM), jnp.float32)],
        interpret=interpret,
    )(embeddings, labels, bonafide_centers)
    return out[0, 0]


# speedup vs baseline: 1.1748x; 1.1748x over previous
import functools

import jax
import jax.numpy as jnp
from jax.experimental import pallas as pl
from jax.experimental.pallas import tpu as pltpu

B = 16384
K = 1024
D = 512
ALPHA = 1.0

BM = 4096
NB = B // BM
TM = 256
NT = BM // TM


def _loss_kernel(emb_ref, lab_ref, cent_ref, out_ref, cn_ref, acc_ref):
    i = pl.program_id(0)

    @pl.when(i == 0)
    def _init():
        c = cent_ref[...]
        cs = jnp.sum(c * c, axis=1, keepdims=True)
        inv = -2.0 / jnp.maximum(jnp.sqrt(cs), 1e-12)
        cn_ref[...] = (c * inv).T
        acc_ref[...] = jnp.zeros((3, TM), jnp.float32)

    cn = cn_ref[...]
    acc_b = acc_ref[0, :]
    acc_s = acc_ref[1, :]
    acc_n = acc_ref[2, :]
    for t in range(NT):
        e = emb_ref[t * TM:(t + 1) * TM, :]
        dot = jax.lax.dot_general(
            e, cn, (((1,), (0,)), ((), ())),
            preferred_element_type=jnp.float32,
        )
        m = jnp.min(dot, axis=1)
        es = jnp.sum(e * e, axis=1)
        inv_e = 1.0 / jnp.maximum(jnp.sqrt(es), 1e-12)
        min_d2 = jnp.maximum(2.0 + m * inv_e, 1e-12)

        lab = lab_ref[t * TM:(t + 1) * TM]
        bona = lab == 0
        acc_b = acc_b + jnp.where(bona, min_d2, 0.0)
        acc_s = acc_s + jnp.where(bona, 0.0, min_d2)
        acc_n = acc_n + lab.astype(jnp.float32)
    acc_ref[0, :] = acc_b
    acc_ref[1, :] = acc_s
    acc_ref[2, :] = acc_n

    @pl.when(i == NB - 1)
    def _finalize():
        n_spoof = jnp.sum(acc_ref[2, :])
        n_bona = float(B) - n_spoof
        bona_loss = jnp.sum(acc_ref[0, :]) / jnp.maximum(n_bona, 1.0)
        spoof_loss = -ALPHA * (jnp.sum(acc_ref[1, :]) / jnp.maximum(n_spoof, 1.0))
        total = (jnp.where(n_bona > 0.0, bona_loss, 0.0)
                 + jnp.where(n_spoof > 0.0, spoof_loss, 0.0))
        out_ref[0, 0] = total


@functools.partial(jax.jit, static_argnames=("interpret",))
def kernel(embeddings, labels, bonafide_centers, interpret=False):
    out = pl.pallas_call(
        _loss_kernel,
        grid=(NB,),
        in_specs=[
            pl.BlockSpec((BM, D), lambda i: (i, 0)),
            pl.BlockSpec((BM,), lambda i: (i,)),
            pl.BlockSpec((K, D), lambda i: (0, 0)),
        ],
        out_specs=pl.BlockSpec(memory_space=pltpu.SMEM),
        out_shape=jax.ShapeDtypeStruct((1, 1), jnp.float32),
        scratch_shapes=[pltpu.VMEM((D, K), jnp.float32),
                        pltpu.VMEM((3, TM), jnp.float32)],
        interpret=interpret,
    )(embeddings, labels, bonafide_centers)
    return out[0, 0]
